# Initial kernel scaffold; baseline (speedup 1.0000x reference)
import jax
import jax.numpy as jnp
from jax import lax
from jax.experimental import pallas as pl
from jax.experimental.pallas import tpu as pltpu
from jax.experimental.pallas import tpu_sc as plsc

N = 10000
E = 320000
F_IN = 128
H = 16
LIN = 100
NUM_GRAPHS = 512
NUM_CLASSES = 12

NTILE = 16
NP = 10240
TN = NP // NTILE
CHUNK = 128
KCH = 8
TE_BLOCKS = 20
EP = NTILE * TE_BLOCKS * KCH * CHUNK
ROWS_PER_TILE = TE_BLOCKS * KCH
GP = 640

_mesh = plsc.VectorSubcoreMesh(
    core_axis_name="c", subcore_axis_name="s", num_cores=2, num_subcores=16
)


def _rsqrt16(v):
    i = lax.bitcast_convert_type(v, jnp.int32)
    i = jnp.int32(0x5F3759DF) - lax.shift_right_arithmetic(i, 1)
    y = lax.bitcast_convert_type(i, jnp.float32)
    for _ in range(3):
        y = y * (1.5 - 0.5 * v * y * y)
    return y


def _bcast_lane(ref, r):
    return plsc.load_gather(ref, [jnp.full((16,), r, jnp.int32)])


def _sc_layer1_body(src_hbm, dst_hbm, h1_hbm, b1_hbm,
                    r1_hbm, dinv_hbm,
                    hs_sh, agg_sh, deg_sh,
                    zbuf, ones, idxa, idxb, rows,
                    h1buf, hsbuf, aggbuf, dinvbuf, degbuf, b1buf,
                    sem, sem2):
    c = lax.axis_index("c")
    t = lax.axis_index("s")

    @pl.when(c == 0)
    def _():
        nbase = t * TN

        for i in range(CHUNK):
            zbuf[i, :] = jnp.zeros((16,), jnp.float32)
        for i in range(KCH):
            ones[pl.ds(i * 16, 16)] = jnp.ones((16,), jnp.float32)
        for k in range(TN // CHUNK):
            pltpu.sync_copy(zbuf, agg_sh.at[pl.ds(nbase + k * CHUNK, CHUNK)])
        for g in range(TN // 16):
            degbuf[pl.ds(g * 16, 16)] = jnp.zeros((16,), jnp.float32)
        pltpu.sync_copy(degbuf, deg_sh.at[pl.ds(nbase, TN)])
        pltpu.sync_copy(b1_hbm, b1buf)
        plsc.subcore_barrier()

        def deg_block(blk, carry):
            r0 = t * ROWS_PER_TILE + blk * KCH
            pltpu.sync_copy(dst_hbm.at[pl.ds(r0, KCH)], idxb)
            hs = [pltpu.async_copy(ones, deg_sh.at[idxb.at[j]], sem, add=True)
                  for j in range(KCH)]
            for h in hs:
                h.wait()
            return carry
        lax.fori_loop(0, TE_BLOCKS, deg_block, 0)
        plsc.subcore_barrier()

        pltpu.sync_copy(deg_sh.at[pl.ds(nbase, TN)], degbuf)
        for g in range(TN // 16):
            v = degbuf[pl.ds(g * 16, 16)] + 1.0
            dinvbuf[pl.ds(g * 16, 16)] = _rsqrt16(v)
        pltpu.sync_copy(dinvbuf, dinv_hbm.at[pl.ds(nbase, TN)])
        pltpu.sync_copy(h1_hbm.at[pl.ds(nbase, TN)], h1buf)

        def scale_row(r, carry):
            hsbuf[r, :] = h1buf[r, :] * _bcast_lane(dinvbuf, r)
            return carry
        lax.fori_loop(0, TN, scale_row, 0)
        pltpu.sync_copy(hsbuf, hs_sh.at[pl.ds(nbase, TN)])
        plsc.subcore_barrier()

        def edge_block(blk, carry):
            r0 = t * ROWS_PER_TILE + blk * KCH
            pltpu.sync_copy(src_hbm.at[pl.ds(r0, KCH)], idxa)
            pltpu.sync_copy(dst_hbm.at[pl.ds(r0, KCH)], idxb)
            ghs = [pltpu.async_copy(hs_sh.at[idxa.at[j]],
                                    rows.at[pl.ds(j * CHUNK, CHUNK)], sem)
                   for j in range(KCH)]
            for h in ghs:
                h.wait()
            shs = [pltpu.async_copy(rows.at[pl.ds(j * CHUNK, CHUNK)],
                                    agg_sh.at[idxb.at[j]], sem2, add=True)
                   for j in range(KCH)]
            for h in shs:
                h.wait()
            return carry
        lax.fori_loop(0, TE_BLOCKS, edge_block, 0)
        plsc.subcore_barrier()

        pltpu.sync_copy(agg_sh.at[pl.ds(nbase, TN)], aggbuf)
        b1v = b1buf[:]

        def comb_row(r, carry):
            v = (aggbuf[r, :] + hsbuf[r, :]) * _bcast_lane(dinvbuf, r) + b1v
            h1buf[r, :] = jnp.maximum(v, 0.0)
            return carry
        lax.fori_loop(0, TN, comb_row, 0)
        pltpu.sync_copy(h1buf, r1_hbm.at[pl.ds(nbase, TN)])


_sc_layer1 = pl.kernel(
    _sc_layer1_body,
    out_type=(jax.ShapeDtypeStruct((NP, H), jnp.float32),
              jax.ShapeDtypeStruct((NP,), jnp.float32)),
    mesh=_mesh,
    scratch_types=[
        pltpu.VMEM_SHARED((NP, H), jnp.float32),
        pltpu.VMEM_SHARED((NP, H), jnp.float32),
        pltpu.VMEM_SHARED((NP,), jnp.float32),
        pltpu.VMEM((CHUNK, H), jnp.float32),
        pltpu.VMEM((CHUNK,), jnp.float32),
        pltpu.VMEM((KCH, CHUNK), jnp.int32),
        pltpu.VMEM((KCH, CHUNK), jnp.int32),
        pltpu.VMEM((KCH * CHUNK, H), jnp.float32),
        pltpu.VMEM((TN, H), jnp.float32),
        pltpu.VMEM((TN, H), jnp.float32),
        pltpu.VMEM((TN, H), jnp.float32),
        pltpu.VMEM((TN,), jnp.float32),
        pltpu.VMEM((TN,), jnp.float32),
        pltpu.VMEM((H,), jnp.float32),
        pltpu.SemaphoreType.DMA,
        pltpu.SemaphoreType.DMA,
    ],
)


def _sc_layer2_body(src_hbm, dst_hbm, hs2_hbm, dinv_hbm, batch_hbm, b2_hbm,
                    pooled_hbm,
                    hs_sh, agg_sh, pool_sh,
                    zbuf, idxa, idxb, bidx, rows,
                    hsbuf, aggbuf, o2buf, dinvbuf, b2buf,
                    sem, sem2):
    c = lax.axis_index("c")
    t = lax.axis_index("s")

    @pl.when(c == 0)
    def _():
        nbase = t * TN

        for i in range(CHUNK):
            zbuf[i, :] = jnp.zeros((16,), jnp.float32)
        for k in range(TN // CHUNK):
            pltpu.sync_copy(zbuf, agg_sh.at[pl.ds(nbase + k * CHUNK, CHUNK)])
        pltpu.sync_copy(zbuf.at[pl.ds(0, GP // NTILE)],
                        pool_sh.at[pl.ds(t * (GP // NTILE), GP // NTILE)])
        pltpu.sync_copy(hs2_hbm.at[pl.ds(nbase, TN)], hsbuf)
        pltpu.sync_copy(hsbuf, hs_sh.at[pl.ds(nbase, TN)])
        pltpu.sync_copy(dinv_hbm.at[pl.ds(nbase, TN)], dinvbuf)
        pltpu.sync_copy(b2_hbm, b2buf)
        plsc.subcore_barrier()

        def edge_block(blk, carry):
            r0 = t * ROWS_PER_TILE + blk * KCH
            pltpu.sync_copy(src_hbm.at[pl.ds(r0, KCH)], idxa)
            pltpu.sync_copy(dst_hbm.at[pl.ds(r0, KCH)], idxb)
            ghs = [pltpu.async_copy(hs_sh.at[idxa.at[j]],
                                    rows.at[pl.ds(j * CHUNK, CHUNK)], sem)
                   for j in range(KCH)]
            for h in ghs:
                h.wait()
            shs = [pltpu.async_copy(rows.at[pl.ds(j * CHUNK, CHUNK)],
                                    agg_sh.at[idxb.at[j]], sem2, add=True)
                   for j in range(KCH)]
            for h in shs:
                h.wait()
            return carry
        lax.fori_loop(0, TE_BLOCKS, edge_block, 0)
        plsc.subcore_barrier()

        pltpu.sync_copy(agg_sh.at[pl.ds(nbase, TN)], aggbuf)
        b2v = b2buf[:]

        def comb_row(r, carry):
            o2buf[r, :] = ((aggbuf[r, :] + hsbuf[r, :])
                           * _bcast_lane(dinvbuf, r) + b2v)
            return carry
        lax.fori_loop(0, TN, comb_row, 0)

        pltpu.sync_copy(batch_hbm.at[pl.ds(t * (TN // CHUNK), TN // CHUNK)],
                        bidx)
        ps = [pltpu.async_copy(o2buf.at[pl.ds(j * CHUNK, CHUNK)],
                               pool_sh.at[bidx.at[j]], sem, add=True)
              for j in range(TN // CHUNK)]
        for h in ps:
            h.wait()
        plsc.subcore_barrier()

        nsg = NUM_GRAPHS // NTILE
        pltpu.sync_copy(pool_sh.at[pl.ds(t * nsg, nsg)],
                        pooled_hbm.at[pl.ds(t * nsg, nsg)])


_sc_layer2 = pl.kernel(
    _sc_layer2_body,
    out_type=jax.ShapeDtypeStruct((NUM_GRAPHS, H), jnp.float32),
    mesh=_mesh,
    scratch_types=[
        pltpu.VMEM_SHARED((NP, H), jnp.float32),
        pltpu.VMEM_SHARED((NP, H), jnp.float32),
        pltpu.VMEM_SHARED((GP, H), jnp.float32),
        pltpu.VMEM((CHUNK, H), jnp.float32),
        pltpu.VMEM((KCH, CHUNK), jnp.int32),
        pltpu.VMEM((KCH, CHUNK), jnp.int32),
        pltpu.VMEM((TN // CHUNK, CHUNK), jnp.int32),
        pltpu.VMEM((KCH * CHUNK, H), jnp.float32),
        pltpu.VMEM((TN, H), jnp.float32),
        pltpu.VMEM((TN, H), jnp.float32),
        pltpu.VMEM((TN, H), jnp.float32),
        pltpu.VMEM((TN,), jnp.float32),
        pltpu.VMEM((H,), jnp.float32),
        pltpu.SemaphoreType.DMA,
        pltpu.SemaphoreType.DMA,
    ],
)


def _tc_matmul1(x, w):
    def body(x_ref, w_ref, o_ref):
        o_ref[:] = jnp.dot(x_ref[:], w_ref[:],
                           preferred_element_type=jnp.float32)
    return pl.pallas_call(
        body,
        grid=(NP // 1024,),
        in_specs=[pl.BlockSpec((1024, F_IN), lambda i: (i, 0)),
                  pl.BlockSpec((F_IN, H), lambda i: (0, 0))],
        out_specs=pl.BlockSpec((1024, H), lambda i: (i, 0)),
        out_shape=jax.ShapeDtypeStruct((NP, H), jnp.float32),
    )(x, w)


def _tc_mid(r1, w2, dinv2d):
    def body(r_ref, w_ref, d_ref, o_ref):
        h2 = jnp.dot(r_ref[:], w_ref[:], preferred_element_type=jnp.float32)
        o_ref[:] = h2 * d_ref[:]
    return pl.pallas_call(
        body,
        grid=(NP // 2048,),
        in_specs=[pl.BlockSpec((2048, H), lambda i: (i, 0)),
                  pl.BlockSpec((H, H), lambda i: (0, 0)),
                  pl.BlockSpec((2048, 1), lambda i: (i, 0))],
        out_specs=pl.BlockSpec((2048, H), lambda i: (i, 0)),
        out_shape=jax.ShapeDtypeStruct((NP, H), jnp.float32),
    )(r1, w2, dinv2d)


def _tc_head(pooled, wl1, bl1, wl2, bl2):
    def body(p_ref, w1_ref, b1_ref, w2_ref, b2_ref, o_ref):
        p = jnp.maximum(p_ref[:], 0.0)
        a = (jnp.dot(p, w1_ref[:], preferred_element_type=jnp.float32)
             + b1_ref[:])
        a = jnp.maximum(a, 0.0)
        o_ref[:] = (jnp.dot(a, w2_ref[:], preferred_element_type=jnp.float32)
                    + b2_ref[:])
    return pl.pallas_call(
        body,
        out_shape=jax.ShapeDtypeStruct((NUM_GRAPHS, NUM_CLASSES), jnp.float32),
    )(pooled, wl1, bl1, wl2, bl2)


def kernel(x, edge_index, batch, W1, b1, W2, b2, Wl1, bl1, Wl2, bl2):
    src = edge_index[0]
    dst = edge_index[1]
    pad_e = EP - E
    pidx = jnp.arange(pad_e, dtype=jnp.int32)
    src_p = jnp.concatenate([src, pidx % 16]).reshape(EP // CHUNK, CHUNK)
    dst_p = jnp.concatenate([dst, N + (pidx % 16)]).reshape(EP // CHUNK, CHUNK)
    pad_n = NP - N
    batch_p = jnp.concatenate(
        [batch, NUM_GRAPHS + (jnp.arange(pad_n, dtype=jnp.int32) % 16)]
    ).reshape(NP // CHUNK, CHUNK)
    x_p = jnp.pad(x, ((0, pad_n), (0, 0)))

    h1 = _tc_matmul1(x_p, W1)
    r1, dinv = _sc_layer1(src_p, dst_p, h1, b1)
    hs2 = _tc_mid(r1, W2, dinv.reshape(NP, 1))
    pooled = _sc_layer2(src_p, dst_p, hs2, dinv, batch_p, b2)
    return _tc_head(pooled, Wl1, bl1.reshape(1, LIN), Wl2,
                    bl2.reshape(1, NUM_CLASSES))


# speedup vs baseline: 47.8529x; 47.8529x over previous
import jax
import jax.numpy as jnp
from jax import lax
from jax.experimental import pallas as pl
from jax.experimental.pallas import tpu as pltpu
from jax.experimental.pallas import tpu_sc as plsc

N = 10000
E = 320000
F_IN = 128
H = 16
LIN = 100
NUM_GRAPHS = 512
NUM_CLASSES = 12

NTILE = 16
NP = 10240
TN = NP // NTILE
CHUNK = 128
KCH = 8
TE_BLOCKS = 20
EP = NTILE * TE_BLOCKS * KCH * CHUNK
ROWS_PER_TILE = TE_BLOCKS * KCH
GP = 640

_mesh = plsc.VectorSubcoreMesh(
    core_axis_name="c", subcore_axis_name="s", num_cores=2, num_subcores=16
)

_sc_params = pltpu.CompilerParams(use_tc_tiling_on_sc=False)


def _rsqrt16(v):
    i = lax.bitcast_convert_type(v, jnp.int32)
    i = jnp.int32(0x5F3759DF) - lax.shift_right_arithmetic(i, 1)
    y = lax.bitcast_convert_type(i, jnp.float32)
    for _ in range(3):
        y = y * (1.5 - 0.5 * v * y * y)
    return y


def _edge_pipeline(src_hbm, dst_hbm, hs_sh, agg_sh, idxa, idxb, rows,
                   sem, sem2, t):
    def pair(k, carry):
        r0 = t * ROWS_PER_TILE + k * (2 * KCH)
        pltpu.sync_copy(src_hbm.at[pl.ds(r0, 2 * KCH)], idxa)
        pltpu.sync_copy(dst_hbm.at[pl.ds(r0, 2 * KCH)], idxb)
        g0 = [pltpu.async_copy(hs_sh.at[idxa.at[j]],
                               rows.at[pl.ds(j * CHUNK, CHUNK)], sem)
              for j in range(KCH)]
        for h in g0:
            h.wait()
        s0 = [pltpu.async_copy(rows.at[pl.ds(j * CHUNK, CHUNK)],
                               agg_sh.at[idxb.at[j]], sem2, add=True)
              for j in range(KCH)]
        g1 = [pltpu.async_copy(hs_sh.at[idxa.at[j]],
                               rows.at[pl.ds(j * CHUNK, CHUNK)], sem)
              for j in range(KCH, 2 * KCH)]
        for h in g1:
            h.wait()
        for h in s0:
            h.wait()
        s1 = [pltpu.async_copy(rows.at[pl.ds(j * CHUNK, CHUNK)],
                               agg_sh.at[idxb.at[j]], sem2, add=True)
              for j in range(KCH, 2 * KCH)]
        for h in s1:
            h.wait()
        return carry
    lax.fori_loop(0, TE_BLOCKS // 2, pair, 0)


def _bcast_lane(ref, r):
    return jnp.full((16,), ref[pl.ds(r, 16)][0], jnp.float32)


def _sc_layer1_body(src_hbm, dst_hbm, h1_hbm, b1_hbm,
                    r1_hbm, dinv_hbm,
                    hs_sh, agg_sh, deg_sh,
                    zbuf, ones, idxa, idxb, rows,
                    h1buf, hsbuf, aggbuf, dinvbuf, degbuf, b1buf,
                    sem, sem2):
    c = lax.axis_index("c")
    t = lax.axis_index("s")

    @pl.when(c == 0)
    def _():
        nbase = t * TN

        for i in range(CHUNK):
            zbuf[i, :] = jnp.zeros((16,), jnp.float32)
        for i in range(KCH):
            ones[pl.ds(i * 16, 16)] = jnp.ones((16,), jnp.float32)
        for k in range(TN // CHUNK):
            pltpu.sync_copy(zbuf, agg_sh.at[pl.ds(nbase + k * CHUNK, CHUNK)])
        for g in range(TN // 16):
            degbuf[pl.ds(g * 16, 16)] = jnp.zeros((16,), jnp.float32)
        pltpu.sync_copy(degbuf, deg_sh.at[pl.ds(nbase, TN)])
        pltpu.sync_copy(b1_hbm, b1buf)
        plsc.subcore_barrier()

        def deg_block(blk, carry):
            r0 = t * ROWS_PER_TILE + blk * (2 * KCH)
            pltpu.sync_copy(dst_hbm.at[pl.ds(r0, 2 * KCH)], idxb)
            hs = [pltpu.async_copy(ones, deg_sh.at[idxb.at[j]], sem, add=True)
                  for j in range(2 * KCH)]
            for h in hs:
                h.wait()
            return carry
        lax.fori_loop(0, TE_BLOCKS // 2, deg_block, 0)
        plsc.subcore_barrier()

        pltpu.sync_copy(deg_sh.at[pl.ds(nbase, TN)], degbuf)
        for g in range(TN // 16):
            v = degbuf[pl.ds(g * 16, 16)] + 1.0
            dinvbuf[pl.ds(g * 16, 16)] = _rsqrt16(v)
        pltpu.sync_copy(dinvbuf.at[pl.ds(0, TN)], dinv_hbm.at[pl.ds(nbase, TN)])
        pltpu.sync_copy(h1_hbm.at[pl.ds(nbase, TN)], h1buf)

        def scale_row(r, carry):
            hsbuf[r, :] = h1buf[r, :] * _bcast_lane(dinvbuf, r)
            return carry
        lax.fori_loop(0, TN, scale_row, 0)
        pltpu.sync_copy(hsbuf, hs_sh.at[pl.ds(nbase, TN)])
        plsc.subcore_barrier()

        _edge_pipeline(src_hbm, dst_hbm, hs_sh, agg_sh, idxa, idxb, rows,
                       sem, sem2, t)
        plsc.subcore_barrier()

        pltpu.sync_copy(agg_sh.at[pl.ds(nbase, TN)], aggbuf)
        b1v = b1buf[:]

        def comb_row(r, carry):
            v = (aggbuf[r, :] + hsbuf[r, :]) * _bcast_lane(dinvbuf, r) + b1v
            h1buf[r, :] = jnp.maximum(v, 0.0)
            return carry
        lax.fori_loop(0, TN, comb_row, 0)
        pltpu.sync_copy(h1buf, r1_hbm.at[pl.ds(nbase, TN)])


_sc_layer1 = pl.kernel(
    _sc_layer1_body,
    out_type=(jax.ShapeDtypeStruct((NP, H), jnp.float32),
              jax.ShapeDtypeStruct((NP,), jnp.float32)),
    mesh=_mesh,
    scratch_types=[
        pltpu.VMEM_SHARED((NP, H), jnp.float32),
        pltpu.VMEM_SHARED((NP, H), jnp.float32),
        pltpu.VMEM_SHARED((NP,), jnp.float32),
        pltpu.VMEM((CHUNK, H), jnp.float32),
        pltpu.VMEM((CHUNK,), jnp.float32),
        pltpu.VMEM((2 * KCH, CHUNK), jnp.int32),
        pltpu.VMEM((2 * KCH, CHUNK), jnp.int32),
        pltpu.VMEM((2 * KCH * CHUNK, H), jnp.float32),
        pltpu.VMEM((TN, H), jnp.float32),
        pltpu.VMEM((TN, H), jnp.float32),
        pltpu.VMEM((TN, H), jnp.float32),
        pltpu.VMEM((TN + 16,), jnp.float32),
        pltpu.VMEM((TN,), jnp.float32),
        pltpu.VMEM((H,), jnp.float32),
        pltpu.SemaphoreType.DMA,
        pltpu.SemaphoreType.DMA,
    ],
    compiler_params=_sc_params,
)


def _sc_layer2_body(src_hbm, dst_hbm, hs2_hbm, dinv_hbm, batch_hbm, b2_hbm,
                    pooled_hbm,
                    hs_sh, agg_sh, pool_sh,
                    zbuf, idxa, idxb, bidx, rows,
                    hsbuf, aggbuf, o2buf, dinvbuf, b2buf,
                    sem, sem2):
    c = lax.axis_index("c")
    t = lax.axis_index("s")

    @pl.when(c == 0)
    def _():
        nbase = t * TN

        for i in range(CHUNK):
            zbuf[i, :] = jnp.zeros((16,), jnp.float32)
        for k in range(TN // CHUNK):
            pltpu.sync_copy(zbuf, agg_sh.at[pl.ds(nbase + k * CHUNK, CHUNK)])
        pltpu.sync_copy(zbuf.at[pl.ds(0, GP // NTILE)],
                        pool_sh.at[pl.ds(t * (GP // NTILE), GP // NTILE)])
        pltpu.sync_copy(hs2_hbm.at[pl.ds(nbase, TN)], hsbuf)
        pltpu.sync_copy(hsbuf, hs_sh.at[pl.ds(nbase, TN)])
        pltpu.sync_copy(dinv_hbm.at[pl.ds(nbase, TN)], dinvbuf.at[pl.ds(0, TN)])
        pltpu.sync_copy(b2_hbm, b2buf)
        plsc.subcore_barrier()

        _edge_pipeline(src_hbm, dst_hbm, hs_sh, agg_sh, idxa, idxb, rows,
                       sem, sem2, t)
        plsc.subcore_barrier()

        pltpu.sync_copy(agg_sh.at[pl.ds(nbase, TN)], aggbuf)
        b2v = b2buf[:]

        def comb_row(r, carry):
            o2buf[r, :] = ((aggbuf[r, :] + hsbuf[r, :])
                           * _bcast_lane(dinvbuf, r) + b2v)
            return carry
        lax.fori_loop(0, TN, comb_row, 0)

        pltpu.sync_copy(batch_hbm.at[pl.ds(t * (TN // CHUNK), TN // CHUNK)],
                        bidx)
        ps = [pltpu.async_copy(o2buf.at[pl.ds(j * CHUNK, CHUNK)],
                               pool_sh.at[bidx.at[j]], sem, add=True)
              for j in range(TN // CHUNK)]
        for h in ps:
            h.wait()
        plsc.subcore_barrier()

        nsg = NUM_GRAPHS // NTILE
        pltpu.sync_copy(pool_sh.at[pl.ds(t * nsg, nsg)],
                        pooled_hbm.at[pl.ds(t * nsg, nsg)])


_sc_layer2 = pl.kernel(
    _sc_layer2_body,
    out_type=jax.ShapeDtypeStruct((NUM_GRAPHS, H), jnp.float32),
    mesh=_mesh,
    scratch_types=[
        pltpu.VMEM_SHARED((NP, H), jnp.float32),
        pltpu.VMEM_SHARED((NP, H), jnp.float32),
        pltpu.VMEM_SHARED((GP, H), jnp.float32),
        pltpu.VMEM((CHUNK, H), jnp.float32),
        pltpu.VMEM((2 * KCH, CHUNK), jnp.int32),
        pltpu.VMEM((2 * KCH, CHUNK), jnp.int32),
        pltpu.VMEM((TN // CHUNK, CHUNK), jnp.int32),
        pltpu.VMEM((2 * KCH * CHUNK, H), jnp.float32),
        pltpu.VMEM((TN, H), jnp.float32),
        pltpu.VMEM((TN, H), jnp.float32),
        pltpu.VMEM((TN, H), jnp.float32),
        pltpu.VMEM((TN + 16,), jnp.float32),
        pltpu.VMEM((H,), jnp.float32),
        pltpu.SemaphoreType.DMA,
        pltpu.SemaphoreType.DMA,
    ],
    compiler_params=_sc_params,
)


def _tc_matmul1(x, w):
    def body(x_ref, w_ref, o_ref):
        o_ref[:] = jnp.dot(x_ref[:], w_ref[:],
                           preferred_element_type=jnp.float32)
    return pl.pallas_call(
        body,
        grid=(NP // 1024,),
        in_specs=[pl.BlockSpec((1024, F_IN), lambda i: (i, 0)),
                  pl.BlockSpec((F_IN, H), lambda i: (0, 0))],
        out_specs=pl.BlockSpec((1024, H), lambda i: (i, 0)),
        out_shape=jax.ShapeDtypeStruct((NP, H), jnp.float32),
    )(x, w)


def _tc_mid(r1, w2, dinv2d):
    def body(r_ref, w_ref, d_ref, o_ref):
        h2 = jnp.dot(r_ref[:], w_ref[:], preferred_element_type=jnp.float32)
        o_ref[:] = h2 * d_ref[:]
    return pl.pallas_call(
        body,
        grid=(NP // 2048,),
        in_specs=[pl.BlockSpec((2048, H), lambda i: (i, 0)),
                  pl.BlockSpec((H, H), lambda i: (0, 0)),
                  pl.BlockSpec((2048, 1), lambda i: (i, 0))],
        out_specs=pl.BlockSpec((2048, H), lambda i: (i, 0)),
        out_shape=jax.ShapeDtypeStruct((NP, H), jnp.float32),
    )(r1, w2, dinv2d)


def _tc_head(pooled, wl1, bl1, wl2, bl2):
    def body(p_ref, w1_ref, b1_ref, w2_ref, b2_ref, o_ref):
        p = jnp.maximum(p_ref[:], 0.0)
        a = (jnp.dot(p, w1_ref[:], preferred_element_type=jnp.float32)
             + b1_ref[:])
        a = jnp.maximum(a, 0.0)
        o_ref[:] = (jnp.dot(a, w2_ref[:], preferred_element_type=jnp.float32)
                    + b2_ref[:])
    return pl.pallas_call(
        body,
        out_shape=jax.ShapeDtypeStruct((NUM_GRAPHS, NUM_CLASSES), jnp.float32),
    )(pooled, wl1, bl1, wl2, bl2)


def kernel(x, edge_index, batch, W1, b1, W2, b2, Wl1, bl1, Wl2, bl2):
    src = edge_index[0]
    dst = edge_index[1]
    pad_e = EP - E
    pidx = jnp.arange(pad_e, dtype=jnp.int32)
    src_p = jnp.concatenate([src, pidx % 16]).reshape(EP // CHUNK, CHUNK)
    dst_p = jnp.concatenate([dst, N + (pidx % 16)]).reshape(EP // CHUNK, CHUNK)
    pad_n = NP - N
    batch_p = jnp.concatenate(
        [batch, NUM_GRAPHS + (jnp.arange(pad_n, dtype=jnp.int32) % 16)]
    ).reshape(NP // CHUNK, CHUNK)
    x_p = jnp.pad(x, ((0, pad_n), (0, 0)))

    h1 = _tc_matmul1(x_p, W1)
    r1, dinv = _sc_layer1(src_p, dst_p, h1, b1)
    hs2 = _tc_mid(r1, W2, dinv.reshape(NP, 1))
    pooled = _sc_layer2(src_p, dst_p, hs2, dinv, batch_p, b2)
    return _tc_head(pooled, Wl1, bl1.reshape(1, LIN), Wl2,
                    bl2.reshape(1, NUM_CLASSES))


# speedup vs baseline: 60.0147x; 1.2541x over previous
import jax
import jax.numpy as jnp
from jax import lax
from jax.experimental import pallas as pl
from jax.experimental.pallas import tpu as pltpu
from jax.experimental.pallas import tpu_sc as plsc

N = 10000
E = 320000
F_IN = 128
H = 16
LIN = 100
NUM_GRAPHS = 512
NUM_CLASSES = 12

NTILE = 16
NP = 10240
TN = NP // NTILE
CHUNK = 128
KCH = 8
TE_BLOCKS = 20
EP = NTILE * TE_BLOCKS * KCH * CHUNK
ROWS_PER_TILE = TE_BLOCKS * KCH
HALF_ROWS = EP // CHUNK // 2
HROWS_PER_TILE = HALF_ROWS // NTILE
GP = 640

_mesh = plsc.VectorSubcoreMesh(
    core_axis_name="c", subcore_axis_name="s", num_cores=2, num_subcores=16
)

_sc_params = pltpu.CompilerParams(use_tc_tiling_on_sc=False)


def _rsqrt16(v):
    i = lax.bitcast_convert_type(v, jnp.int32)
    i = jnp.int32(0x5F3759DF) - lax.shift_right_arithmetic(i, 1)
    y = lax.bitcast_convert_type(i, jnp.float32)
    for _ in range(3):
        y = y * (1.5 - 0.5 * v * y * y)
    return y


def _bcast_lane(ref, r):
    return jnp.full((16,), ref[pl.ds(r, 16)][0], jnp.float32)


def _edge_pipeline(src_hbm, dst_hbm, hs_sh, agg_sh, idxa, idxb, rows,
                   sem, sem2, base_row, n_pairs):
    def pair(k, carry):
        r0 = base_row + k * (2 * KCH)
        pltpu.sync_copy(src_hbm.at[pl.ds(r0, 2 * KCH)], idxa)
        pltpu.sync_copy(dst_hbm.at[pl.ds(r0, 2 * KCH)], idxb)
        g0 = [pltpu.async_copy(hs_sh.at[idxa.at[j]],
                               rows.at[pl.ds(j * CHUNK, CHUNK)], sem)
              for j in range(KCH)]
        for h in g0:
            h.wait()
        s0 = [pltpu.async_copy(rows.at[pl.ds(j * CHUNK, CHUNK)],
                               agg_sh.at[idxb.at[j]], sem2, add=True)
              for j in range(KCH)]
        g1 = [pltpu.async_copy(hs_sh.at[idxa.at[j]],
                               rows.at[pl.ds(j * CHUNK, CHUNK)], sem)
              for j in range(KCH, 2 * KCH)]
        for h in g1:
            h.wait()
        for h in s0:
            h.wait()
        s1 = [pltpu.async_copy(rows.at[pl.ds(j * CHUNK, CHUNK)],
                               agg_sh.at[idxb.at[j]], sem2, add=True)
              for j in range(KCH, 2 * KCH)]
        for h in s1:
            h.wait()
        return carry
    lax.fori_loop(0, n_pairs, pair, 0)


def _zero_shared(zbuf, agg_sh, nbase):
    for k in range(TN // CHUNK):
        pltpu.sync_copy(zbuf, agg_sh.at[pl.ds(nbase + k * CHUNK, CHUNK)])


def _sc_layer1_body(src_hbm, dst_hbm, h1_hbm,
                    agg_a_hbm, agg_b_hbm, dinv_hbm,
                    hs_sh, agg_sh, deg_sh,
                    zbuf, ones, idxa, idxb, rows,
                    h1buf, hsbuf, dinvbuf, degbuf,
                    sem, sem2):
    c = lax.axis_index("c")
    t = lax.axis_index("s")
    nbase = t * TN

    for i in range(CHUNK):
        zbuf[i, :] = jnp.zeros((16,), jnp.float32)
    for i in range(KCH):
        ones[pl.ds(i * 16, 16)] = jnp.ones((16,), jnp.float32)
    _zero_shared(zbuf, agg_sh, nbase)
    for g in range(TN // 16):
        degbuf[pl.ds(g * 16, 16)] = jnp.zeros((16,), jnp.float32)
    pltpu.sync_copy(degbuf, deg_sh.at[pl.ds(nbase, TN)])
    pltpu.sync_copy(h1_hbm.at[pl.ds(nbase, TN)], h1buf)
    plsc.subcore_barrier()

    def deg_block(blk, carry):
        r0 = t * ROWS_PER_TILE + blk * (2 * KCH)
        pltpu.sync_copy(dst_hbm.at[pl.ds(r0, 2 * KCH)], idxb)
        hs = [pltpu.async_copy(ones, deg_sh.at[idxb.at[j]], sem, add=True)
              for j in range(2 * KCH)]
        for h in hs:
            h.wait()
        return carry
    lax.fori_loop(0, TE_BLOCKS // 2, deg_block, 0)
    plsc.subcore_barrier()

    pltpu.sync_copy(deg_sh.at[pl.ds(nbase, TN)], degbuf)
    for g in range(TN // 16):
        v = degbuf[pl.ds(g * 16, 16)] + 1.0
        dinvbuf[pl.ds(g * 16, 16)] = _rsqrt16(v)

    @pl.when(c == 0)
    def _():
        pltpu.sync_copy(dinvbuf.at[pl.ds(0, TN)], dinv_hbm.at[pl.ds(nbase, TN)])

    def scale_row(r, carry):
        hsbuf[r, :] = h1buf[r, :] * _bcast_lane(dinvbuf, r)
        return carry
    lax.fori_loop(0, TN, scale_row, 0)
    pltpu.sync_copy(hsbuf, hs_sh.at[pl.ds(nbase, TN)])
    plsc.subcore_barrier()

    _edge_pipeline(src_hbm, dst_hbm, hs_sh, agg_sh, idxa, idxb, rows,
                   sem, sem2,
                   c * HALF_ROWS + t * HROWS_PER_TILE, HROWS_PER_TILE // 16)
    plsc.subcore_barrier()

    pltpu.sync_copy(agg_sh.at[pl.ds(nbase, TN)], h1buf)

    @pl.when(c == 0)
    def _():
        pltpu.sync_copy(h1buf, agg_a_hbm.at[pl.ds(nbase, TN)])

    @pl.when(c == 1)
    def _():
        pltpu.sync_copy(h1buf, agg_b_hbm.at[pl.ds(nbase, TN)])


_sc_layer1 = pl.kernel(
    _sc_layer1_body,
    out_type=(jax.ShapeDtypeStruct((NP, H), jnp.float32),
              jax.ShapeDtypeStruct((NP, H), jnp.float32),
              jax.ShapeDtypeStruct((NP,), jnp.float32)),
    mesh=_mesh,
    scratch_types=[
        pltpu.VMEM_SHARED((NP, H), jnp.float32),
        pltpu.VMEM_SHARED((NP, H), jnp.float32),
        pltpu.VMEM_SHARED((NP,), jnp.float32),
        pltpu.VMEM((CHUNK, H), jnp.float32),
        pltpu.VMEM((CHUNK,), jnp.float32),
        pltpu.VMEM((2 * KCH, CHUNK), jnp.int32),
        pltpu.VMEM((2 * KCH, CHUNK), jnp.int32),
        pltpu.VMEM((2 * KCH * CHUNK, H), jnp.float32),
        pltpu.VMEM((TN, H), jnp.float32),
        pltpu.VMEM((TN, H), jnp.float32),
        pltpu.VMEM((TN + 16,), jnp.float32),
        pltpu.VMEM((TN,), jnp.float32),
        pltpu.SemaphoreType.DMA,
        pltpu.SemaphoreType.DMA,
    ],
    compiler_params=_sc_params,
)


def _sc_layer2_body(src_hbm, dst_hbm, hs2_hbm, dinv_hbm, batch_hbm, b2_hbm,
                    pooled_a_hbm, pooled_b_hbm,
                    hs_sh, agg_sh, pool_sh,
                    zbuf, idxa, idxb, bidx, rows,
                    hsbuf, aggbuf, o2buf, dinvbuf, b2buf,
                    sem, sem2):
    c = lax.axis_index("c")
    t = lax.axis_index("s")
    nbase = t * TN

    for i in range(CHUNK):
        zbuf[i, :] = jnp.zeros((16,), jnp.float32)
    _zero_shared(zbuf, agg_sh, nbase)
    pltpu.sync_copy(zbuf.at[pl.ds(0, GP // NTILE)],
                    pool_sh.at[pl.ds(t * (GP // NTILE), GP // NTILE)])
    pltpu.sync_copy(hs2_hbm.at[pl.ds(nbase, TN)], hsbuf)
    pltpu.sync_copy(hsbuf, hs_sh.at[pl.ds(nbase, TN)])
    pltpu.sync_copy(dinv_hbm.at[pl.ds(nbase, TN)], dinvbuf.at[pl.ds(0, TN)])
    pltpu.sync_copy(b2_hbm, b2buf)
    plsc.subcore_barrier()

    _edge_pipeline(src_hbm, dst_hbm, hs_sh, agg_sh, idxa, idxb, rows,
                   sem, sem2,
                   c * HALF_ROWS + t * HROWS_PER_TILE, HROWS_PER_TILE // 16)
    plsc.subcore_barrier()

    pltpu.sync_copy(agg_sh.at[pl.ds(nbase, TN)], aggbuf)
    b2v = b2buf[:]
    mval = jnp.where(c == 0, 1.0, 0.0).astype(jnp.float32)

    def comb_row(r, carry):
        o2buf[r, :] = ((aggbuf[r, :] + hsbuf[r, :] * mval)
                       * _bcast_lane(dinvbuf, r) + b2v * mval)
        return carry
    lax.fori_loop(0, TN, comb_row, 0)

    pltpu.sync_copy(batch_hbm.at[pl.ds(t * (TN // CHUNK), TN // CHUNK)],
                    bidx)
    ps = [pltpu.async_copy(o2buf.at[pl.ds(j * CHUNK, CHUNK)],
                           pool_sh.at[bidx.at[j]], sem, add=True)
          for j in range(TN // CHUNK)]
    for h in ps:
        h.wait()
    plsc.subcore_barrier()

    nsg = NUM_GRAPHS // NTILE

    @pl.when(c == 0)
    def _():
        pltpu.sync_copy(pool_sh.at[pl.ds(t * nsg, nsg)],
                        pooled_a_hbm.at[pl.ds(t * nsg, nsg)])

    @pl.when(c == 1)
    def _():
        pltpu.sync_copy(pool_sh.at[pl.ds(t * nsg, nsg)],
                        pooled_b_hbm.at[pl.ds(t * nsg, nsg)])


_sc_layer2 = pl.kernel(
    _sc_layer2_body,
    out_type=(jax.ShapeDtypeStruct((NUM_GRAPHS, H), jnp.float32),
              jax.ShapeDtypeStruct((NUM_GRAPHS, H), jnp.float32)),
    mesh=_mesh,
    scratch_types=[
        pltpu.VMEM_SHARED((NP, H), jnp.float32),
        pltpu.VMEM_SHARED((NP, H), jnp.float32),
        pltpu.VMEM_SHARED((GP, H), jnp.float32),
        pltpu.VMEM((CHUNK, H), jnp.float32),
        pltpu.VMEM((2 * KCH, CHUNK), jnp.int32),
        pltpu.VMEM((2 * KCH, CHUNK), jnp.int32),
        pltpu.VMEM((TN // CHUNK, CHUNK), jnp.int32),
        pltpu.VMEM((2 * KCH * CHUNK, H), jnp.float32),
        pltpu.VMEM((TN, H), jnp.float32),
        pltpu.VMEM((TN, H), jnp.float32),
        pltpu.VMEM((TN, H), jnp.float32),
        pltpu.VMEM((TN + 16,), jnp.float32),
        pltpu.VMEM((H,), jnp.float32),
        pltpu.SemaphoreType.DMA,
        pltpu.SemaphoreType.DMA,
    ],
    compiler_params=_sc_params,
)


def _tc_matmul1(x, w):
    def body(x_ref, w_ref, o_ref):
        o_ref[:] = jnp.dot(x_ref[:], w_ref[:],
                           preferred_element_type=jnp.float32)
    return pl.pallas_call(
        body,
        grid=(N // 1000,),
        in_specs=[pl.BlockSpec((1000, F_IN), lambda i: (i, 0)),
                  pl.BlockSpec((F_IN, H), lambda i: (0, 0))],
        out_specs=pl.BlockSpec((1000, H), lambda i: (i, 0)),
        out_shape=jax.ShapeDtypeStruct((N, H), jnp.float32),
    )(x, w)


def _tc_mid(agg_a, agg_b, h1, w2, b1, dinv2d):
    def body(a_ref, b_ref, h_ref, w_ref, b1_ref, d_ref, o_ref):
        d = d_ref[:]
        r1 = (a_ref[:] + b_ref[:] + h_ref[:] * d) * d + b1_ref[:]
        r1 = jnp.maximum(r1, 0.0)
        h2 = jnp.dot(r1, w_ref[:], preferred_element_type=jnp.float32)
        o_ref[:] = h2 * d
    return pl.pallas_call(
        body,
        grid=(NP // 2048,),
        in_specs=[pl.BlockSpec((2048, H), lambda i: (i, 0)),
                  pl.BlockSpec((2048, H), lambda i: (i, 0)),
                  pl.BlockSpec((2048, H), lambda i: (i, 0)),
                  pl.BlockSpec((H, H), lambda i: (0, 0)),
                  pl.BlockSpec((1, H), lambda i: (0, 0)),
                  pl.BlockSpec((2048, 1), lambda i: (i, 0))],
        out_specs=pl.BlockSpec((2048, H), lambda i: (i, 0)),
        out_shape=jax.ShapeDtypeStruct((NP, H), jnp.float32),
    )(agg_a, agg_b, h1, w2, b1, dinv2d)


def _tc_head(pooled_a, pooled_b, wl1, bl1, wl2, bl2):
    def body(pa_ref, pb_ref, w1_ref, b1_ref, w2_ref, b2_ref, o_ref):
        p = jnp.maximum(pa_ref[:] + pb_ref[:], 0.0)
        a = (jnp.dot(p, w1_ref[:], preferred_element_type=jnp.float32)
             + b1_ref[:])
        a = jnp.maximum(a, 0.0)
        o_ref[:] = (jnp.dot(a, w2_ref[:], preferred_element_type=jnp.float32)
                    + b2_ref[:])
    return pl.pallas_call(
        body,
        out_shape=jax.ShapeDtypeStruct((NUM_GRAPHS, NUM_CLASSES), jnp.float32),
    )(pooled_a, pooled_b, wl1, bl1, wl2, bl2)


def kernel(x, edge_index, batch, W1, b1, W2, b2, Wl1, bl1, Wl2, bl2):
    src = edge_index[0]
    dst = edge_index[1]
    pad_e = EP - E
    pidx = jnp.arange(pad_e, dtype=jnp.int32)
    src_p = jnp.concatenate([src, pidx % 16]).reshape(EP // CHUNK, CHUNK)
    dst_p = jnp.concatenate([dst, N + (pidx % 16)]).reshape(EP // CHUNK, CHUNK)
    pad_n = NP - N
    batch_p = jnp.concatenate(
        [batch, NUM_GRAPHS + (jnp.arange(pad_n, dtype=jnp.int32) % 16)]
    ).reshape(NP // CHUNK, CHUNK)

    h1 = jnp.pad(_tc_matmul1(x, W1), ((0, pad_n), (0, 0)))
    agg_a, agg_b, dinv = _sc_layer1(src_p, dst_p, h1)
    hs2 = _tc_mid(agg_a, agg_b, h1, W2, b1.reshape(1, H), dinv.reshape(NP, 1))
    pooled_a, pooled_b = _sc_layer2(src_p, dst_p, hs2, dinv, batch_p, b2)
    return _tc_head(pooled_a, pooled_b, Wl1, bl1.reshape(1, LIN), Wl2,
                    bl2.reshape(1, NUM_CLASSES))
